# Initial kernel scaffold; baseline (speedup 1.0000x reference)
import functools

import jax
import jax.numpy as jnp
from jax import lax
from jax.experimental import pallas as pl
from jax.experimental.pallas import tpu as pltpu
from jax.experimental.pallas import tpu_sc as plsc

N = 10000
E = 320000
D_IN = 128
HID = 128
HEADS = 8
CLASSES = 40
NEG = 0.2

CB1 = HEADS * HID + 16
CB2 = 48
NBLK = 1000
_INTERP = False


def _lrelu(x):
    return jnp.where(x > 0, x, NEG * x)


def _elu(x):
    return jnp.where(x > 0, x, jnp.expm1(jnp.minimum(x, 0.0)))


def _k1_body(x_ref, w_ref, a_ref, xw_ref, t_ref, m_ref):
    i = pl.program_id(0)
    xw = jnp.dot(x_ref[...], w_ref[...], preferred_element_type=jnp.float32)
    t = jnp.dot(xw, a_ref[...], preferred_element_type=jnp.float32)
    xw_ref[...] = jnp.concatenate(
        [xw, jnp.ones((xw.shape[0], 16), jnp.float32)], axis=1)
    t_ref[...] = t
    tmax = jnp.max(t, axis=0, keepdims=True)

    @pl.when(i == 0)
    def _():
        m_ref[...] = jnp.full((1, 16), -jnp.inf, jnp.float32)

    m_ref[...] = jnp.maximum(m_ref[...], tmax)


def _k1(x, W1, A1):
    grid = (N // NBLK,)
    return pl.pallas_call(
        _k1_body,
        grid=grid,
        in_specs=[
            pl.BlockSpec((NBLK, D_IN), lambda i: (i, 0)),
            pl.BlockSpec((D_IN, HEADS * HID), lambda i: (0, 0)),
            pl.BlockSpec((HEADS * HID, 16), lambda i: (0, 0)),
        ],
        out_specs=[
            pl.BlockSpec((NBLK, CB1), lambda i: (i, 0)),
            pl.BlockSpec((NBLK, 16), lambda i: (i, 0)),
            pl.BlockSpec((1, 16), lambda i: (0, 0)),
        ],
        out_shape=[
            jax.ShapeDtypeStruct((N, CB1), jnp.float32),
            jax.ShapeDtypeStruct((N, 16), jnp.float32),
            jax.ShapeDtypeStruct((1, 16), jnp.float32),
        ],
        interpret=_INTERP,
    )(x, W1, A1)


def _k4_body(p0, p1, d0, d1, xw, ts, td, ms, md, b, out):
    wself = jnp.exp(_lrelu(ts[...] + td[...]) - _lrelu(ms[...] + md[...]))
    den = d0[0] + d1[0] + wself
    num = p0[0] + p1[0] + wself * xw[...]
    out[0] = _elu(num / den + b[...])


def _k4(num_p, xw_ext, T1, maxT, bias1r):
    grid = (N // NBLK, HEADS)
    return pl.pallas_call(
        _k4_body,
        grid=grid,
        in_specs=[
            pl.BlockSpec((1, NBLK, HID), lambda i, h: (0, i, h)),
            pl.BlockSpec((1, NBLK, HID), lambda i, h: (1, i, h)),
            pl.BlockSpec((1, NBLK, 1), lambda i, h: (0, i, HEADS * HID + h)),
            pl.BlockSpec((1, NBLK, 1), lambda i, h: (1, i, HEADS * HID + h)),
            pl.BlockSpec((NBLK, HID), lambda i, h: (i, h)),
            pl.BlockSpec((NBLK, 1), lambda i, h: (i, h)),
            pl.BlockSpec((NBLK, 1), lambda i, h: (i, HEADS + h)),
            pl.BlockSpec((1, 1), lambda i, h: (0, h)),
            pl.BlockSpec((1, 1), lambda i, h: (0, HEADS + h)),
            pl.BlockSpec((1, HID), lambda i, h: (h, 0)),
        ],
        out_specs=pl.BlockSpec((1, NBLK, HID), lambda i, h: (h, i, 0)),
        out_shape=jax.ShapeDtypeStruct((HEADS, N, HID), jnp.float32),
        interpret=_INTERP,
    )(num_p, num_p, num_p, num_p, xw_ext, T1, T1, maxT, maxT, bias1r)


def _k5_body(h_ref, w_ref, a_ref, xw_ref, t_ref, m_ref):
    i = pl.program_id(0)
    acc = jnp.zeros((NBLK, CB2), jnp.float32)
    for hh in range(HEADS):
        acc = acc + jnp.dot(h_ref[hh], w_ref[hh],
                            preferred_element_type=jnp.float32)
    col = lax.broadcasted_iota(jnp.int32, (NBLK, CB2), 1)
    xw2 = jnp.where(col == CLASSES, 1.0, acc)
    t = jnp.dot(xw2, a_ref[...], preferred_element_type=jnp.float32)
    xw_ref[...] = xw2
    t_ref[...] = t
    tmax = jnp.max(t, axis=0, keepdims=True)

    @pl.when(i == 0)
    def _():
        m_ref[...] = jnp.full((1, 16), -jnp.inf, jnp.float32)

    m_ref[...] = jnp.maximum(m_ref[...], tmax)


def _k5(h_arr, W2r, A2):
    grid = (N // NBLK,)
    return pl.pallas_call(
        _k5_body,
        grid=grid,
        in_specs=[
            pl.BlockSpec((HEADS, NBLK, HID), lambda i: (0, i, 0)),
            pl.BlockSpec((HEADS, HID, CB2), lambda i: (0, 0, 0)),
            pl.BlockSpec((CB2, 16), lambda i: (0, 0)),
        ],
        out_specs=[
            pl.BlockSpec((NBLK, CB2), lambda i: (i, 0)),
            pl.BlockSpec((NBLK, 16), lambda i: (i, 0)),
            pl.BlockSpec((1, 16), lambda i: (0, 0)),
        ],
        out_shape=[
            jax.ShapeDtypeStruct((N, CB2), jnp.float32),
            jax.ShapeDtypeStruct((N, 16), jnp.float32),
            jax.ShapeDtypeStruct((1, 16), jnp.float32),
        ],
        interpret=_INTERP,
    )(h_arr, W2r, A2)


def _k8_body(p0, p1, xw, t2, m2, b2, out):
    wself = jnp.exp(_lrelu(t2[:, 0:1] + t2[:, 1:2])
                    - _lrelu(m2[0, 0] + m2[0, 1]))
    s = p0[0] + p1[0] + wself * xw[...]
    den = s[:, CLASSES:CLASSES + 1]
    z = s[:, :CLASSES] / den + b2[:, :CLASSES]
    zmax = jnp.max(z, axis=1, keepdims=True)
    zs = z - zmax
    out[...] = zs - jnp.log(jnp.sum(jnp.exp(zs), axis=1, keepdims=True))


def _k8(num2_p, xw2_ext, T2, maxT2, bias2p):
    grid = (N // NBLK,)
    return pl.pallas_call(
        _k8_body,
        grid=grid,
        in_specs=[
            pl.BlockSpec((1, NBLK, CB2), lambda i: (0, i, 0)),
            pl.BlockSpec((1, NBLK, CB2), lambda i: (1, i, 0)),
            pl.BlockSpec((NBLK, CB2), lambda i: (i, 0)),
            pl.BlockSpec((NBLK, 16), lambda i: (i, 0)),
            pl.BlockSpec((1, 16), lambda i: (0, 0)),
            pl.BlockSpec((1, CB2), lambda i: (0, 0)),
        ],
        out_specs=pl.BlockSpec((NBLK, CLASSES), lambda i: (i, 0)),
        out_shape=jax.ShapeDtypeStruct((N, CLASSES), jnp.float32),
        interpret=_INTERP,
    )(num2_p, num2_p, xw2_ext, T2, maxT2, bias2p)


def _edges_l1(xw_ext, T1, maxT, src, dst):
    M = _lrelu(maxT[0, :HEADS] + maxT[0, HEADS:])
    alpha = _lrelu(T1[src, :HEADS] + T1[dst, HEADS:])
    w = jnp.exp(alpha - M[None, :])
    mult = jnp.concatenate(
        [jnp.repeat(w, HID, axis=1), w, jnp.zeros((E, 8), jnp.float32)],
        axis=1)
    num = jax.ops.segment_sum(mult * xw_ext[src], dst, num_segments=N)
    return jnp.stack([num, jnp.zeros_like(num)])


def _edges_l2(xw2_ext, T2, maxT2, src, dst):
    M = _lrelu(maxT2[0, 0] + maxT2[0, 1])
    alpha = _lrelu(T2[src, 0] + T2[dst, 1])
    w = jnp.exp(alpha - M)
    num = jax.ops.segment_sum(w[:, None] * xw2_ext[src], dst, num_segments=N)
    return jnp.stack([num, jnp.zeros_like(num)])


def kernel(x, edge_index, W1, att_src1, att_dst1, bias1,
           W2, att_src2, att_dst2, bias2):
    src = edge_index[0]
    dst = edge_index[1]

    A1 = jnp.zeros((HEADS * HID, 16), jnp.float32)
    for h in range(HEADS):
        A1 = A1.at[h * HID:(h + 1) * HID, h].set(att_src1[h])
        A1 = A1.at[h * HID:(h + 1) * HID, HEADS + h].set(att_dst1[h])
    W2e = jnp.zeros((HEADS * HID, CB2), jnp.float32).at[:, :CLASSES].set(W2)
    W2r = W2e.reshape(HEADS, HID, CB2)
    A2 = jnp.zeros((CB2, 16), jnp.float32)
    A2 = A2.at[:CLASSES, 0].set(att_src2[0])
    A2 = A2.at[:CLASSES, 1].set(att_dst2[0])
    bias1r = bias1.reshape(HEADS, HID)
    bias2p = jnp.zeros((1, CB2), jnp.float32).at[0, :CLASSES].set(bias2)

    xw_ext, T1, maxT = _k1(x, W1, A1)
    num_p = _edges_l1(xw_ext, T1, maxT, src, dst)
    h_arr = _k4(num_p, xw_ext, T1, maxT, bias1r)
    xw2_ext, T2, maxT2 = _k5(h_arr, W2r, A2)
    num2_p = _edges_l2(xw2_ext, T2, maxT2, src, dst)
    return _k8(num2_p, xw2_ext, T2, maxT2, bias2p)


# speedup vs baseline: 17.4407x; 17.4407x over previous
import functools

import jax
import jax.numpy as jnp
from jax import lax
from jax.experimental import pallas as pl
from jax.experimental.pallas import tpu as pltpu
from jax.experimental.pallas import tpu_sc as plsc

N = 10000
E = 320000
D_IN = 128
HID = 128
HEADS = 8
CLASSES = 40
NEG = 0.2

NTILE1 = 9
CB1 = NTILE1 * HID
CB2 = 128
CB2S = 48
NBLK = 1000
_INTERP = False


def _lrelu(x):
    return jnp.where(x > 0, x, NEG * x)


def _elu(x):
    return jnp.where(x > 0, x, jnp.exp(jnp.minimum(x, 0.0)) - 1.0)


def _k1_body(x_ref, w_ref, a_ref, xw_ref, t_ref, m_ref):
    i = pl.program_id(0)
    xw = jnp.dot(x_ref[...], w_ref[...], preferred_element_type=jnp.float32)
    t = jnp.dot(xw, a_ref[...], preferred_element_type=jnp.float32)
    for tt in range(HEADS):
        xw_ref[tt] = xw[:, tt * HID:(tt + 1) * HID]
    xw_ref[HEADS] = jnp.concatenate(
        [jnp.ones((xw.shape[0], 8), jnp.float32), t[:, :HEADS],
         jnp.zeros((xw.shape[0], HID - 16), jnp.float32)], axis=1)
    t_ref[...] = t
    tmax = jnp.max(t, axis=0, keepdims=True)

    @pl.when(i == 0)
    def _():
        m_ref[...] = jnp.full((1, 16), -jnp.inf, jnp.float32)

    m_ref[...] = jnp.maximum(m_ref[...], tmax)


def _k1(x, W1, A1):
    grid = (N // NBLK,)
    return pl.pallas_call(
        _k1_body,
        grid=grid,
        in_specs=[
            pl.BlockSpec((NBLK, D_IN), lambda i: (i, 0)),
            pl.BlockSpec((D_IN, HEADS * HID), lambda i: (0, 0)),
            pl.BlockSpec((HEADS * HID, 16), lambda i: (0, 0)),
        ],
        out_specs=[
            pl.BlockSpec((NTILE1, NBLK, HID), lambda i: (0, i, 0)),
            pl.BlockSpec((NBLK, 16), lambda i: (i, 0)),
            pl.BlockSpec((1, 16), lambda i: (0, 0)),
        ],
        out_shape=[
            jax.ShapeDtypeStruct((NTILE1, N, HID), jnp.float32),
            jax.ShapeDtypeStruct((N, 16), jnp.float32),
            jax.ShapeDtypeStruct((1, 16), jnp.float32),
        ],
        interpret=_INTERP,
    )(x, W1, A1)


def _k4_body(p0, p1, xw, t, m, b, out):
    wself = jnp.exp(_lrelu(t[:, :HEADS] + t[:, HEADS:])
                    - _lrelu(m[:, :HEADS] + m[:, HEADS:]))
    s3 = p0[0] + p1[0]
    den8 = s3[HEADS]
    for h in range(HEADS):
        ws = wself[:, h:h + 1]
        seg = s3[h] + ws * xw[h]
        den = den8[:, h:h + 1] + ws
        out[h] = _elu(seg / den + b[h:h + 1, :])


def _k4(num_p, xw_ext, T1, maxT, bias1r):
    grid = (N // NBLK,)
    return pl.pallas_call(
        _k4_body,
        grid=grid,
        in_specs=[
            pl.BlockSpec((1, NTILE1, NBLK, HID), lambda i: (0, 0, i, 0)),
            pl.BlockSpec((1, NTILE1, NBLK, HID), lambda i: (1, 0, i, 0)),
            pl.BlockSpec((NTILE1, NBLK, HID), lambda i: (0, i, 0)),
            pl.BlockSpec((NBLK, 16), lambda i: (i, 0)),
            pl.BlockSpec((1, 16), lambda i: (0, 0)),
            pl.BlockSpec((HEADS, HID), lambda i: (0, 0)),
        ],
        out_specs=pl.BlockSpec((HEADS, NBLK, HID), lambda i: (0, i, 0)),
        out_shape=jax.ShapeDtypeStruct((HEADS, N, HID), jnp.float32),
        interpret=_INTERP,
    )(num_p, num_p, xw_ext, T1, maxT, bias1r)


def _k5_body(h_ref, w_ref, a_ref, xw_ref, t_ref, m_ref):
    i = pl.program_id(0)
    acc = jnp.zeros((NBLK, CB2), jnp.float32)
    for hh in range(HEADS):
        acc = acc + jnp.dot(h_ref[hh], w_ref[hh],
                            preferred_element_type=jnp.float32)
    col = lax.broadcasted_iota(jnp.int32, (NBLK, CB2), 1)
    xw2 = jnp.where(col == CLASSES, 1.0, acc)
    t = jnp.dot(xw2, a_ref[...], preferred_element_type=jnp.float32)
    xw_ref[...] = jnp.where(col == CLASSES + 1, t[:, 0:1], xw2)
    t_ref[...] = t
    tmax = jnp.max(t, axis=0, keepdims=True)

    @pl.when(i == 0)
    def _():
        m_ref[...] = jnp.full((1, 16), -jnp.inf, jnp.float32)

    m_ref[...] = jnp.maximum(m_ref[...], tmax)


def _k5(h_arr, W2r, A2):
    grid = (N // NBLK,)
    return pl.pallas_call(
        _k5_body,
        grid=grid,
        in_specs=[
            pl.BlockSpec((HEADS, NBLK, HID), lambda i: (0, i, 0)),
            pl.BlockSpec((HEADS, HID, CB2), lambda i: (0, 0, 0)),
            pl.BlockSpec((CB2, 16), lambda i: (0, 0)),
        ],
        out_specs=[
            pl.BlockSpec((NBLK, CB2), lambda i: (i, 0)),
            pl.BlockSpec((NBLK, 16), lambda i: (i, 0)),
            pl.BlockSpec((1, 16), lambda i: (0, 0)),
        ],
        out_shape=[
            jax.ShapeDtypeStruct((N, CB2), jnp.float32),
            jax.ShapeDtypeStruct((N, 16), jnp.float32),
            jax.ShapeDtypeStruct((1, 16), jnp.float32),
        ],
        interpret=_INTERP,
    )(h_arr, W2r, A2)


def _k8_body(p0, p1, xw, t2, m2, b2, out):
    wself = jnp.exp(_lrelu(t2[:, 0:1] + t2[:, 1:2])
                    - _lrelu(m2[0, 0] + m2[0, 1]))
    s = p0[0] + p1[0] + wself * xw[...]
    den = s[:, CLASSES:CLASSES + 1]
    z = s[:, :CLASSES] / den + b2[:, :CLASSES]
    zmax = jnp.max(z, axis=1, keepdims=True)
    zs = z - zmax
    out[...] = zs - jnp.log(jnp.sum(jnp.exp(zs), axis=1, keepdims=True))


def _k8(num2_p, xw2_ext, T2, maxT2, bias2p):
    grid = (N // NBLK,)
    return pl.pallas_call(
        _k8_body,
        grid=grid,
        in_specs=[
            pl.BlockSpec((1, NBLK, CB2), lambda i: (0, i, 0)),
            pl.BlockSpec((1, NBLK, CB2), lambda i: (1, i, 0)),
            pl.BlockSpec((NBLK, CB2), lambda i: (i, 0)),
            pl.BlockSpec((NBLK, 16), lambda i: (i, 0)),
            pl.BlockSpec((1, 16), lambda i: (0, 0)),
            pl.BlockSpec((1, CB2), lambda i: (0, 0)),
        ],
        out_specs=pl.BlockSpec((NBLK, CLASSES), lambda i: (i, 0)),
        out_shape=jax.ShapeDtypeStruct((N, CLASSES), jnp.float32),
        interpret=_INTERP,
    )(num2_p, num2_p, xw2_ext, T2, maxT2, bias2p)


_B2 = 80
_NT = 32
_EPT = E // _NT
NP = 10240
_RPT = NP // 16


def _iota16():
    return lax.broadcasted_iota(jnp.int32, (16,), 0)


def _splat(v, dtype=jnp.float32):
    return jnp.full((16,), v, dtype)


def _edges_l2_sc(xw2_ext, ad2, maxT2, src, dst, zeros_cb2):
    mesh = plsc.VectorSubcoreMesh(core_axis_name="c", subcore_axis_name="s")

    @functools.partial(
        pl.kernel,
        out_type=jax.ShapeDtypeStruct((2, NP, CB2), jnp.float32),
        mesh=mesh,
        compiler_params=pltpu.CompilerParams(needs_layout_passes=False),
        scratch_types=[
            pltpu.VMEM_SHARED((NP, CB2), jnp.float32),
            pltpu.VMEM((_B2,), jnp.int32),
            pltpu.VMEM((_B2,), jnp.int32),
            pltpu.VMEM((_B2, CB2), jnp.float32),
            pltpu.VMEM((NP,), jnp.float32),
            pltpu.VMEM((16,), jnp.float32),
            pltpu.SemaphoreType.DMA,
        ],
    )
    def k7(xw_hbm, ad2_hbm, mt_hbm, src_hbm, dst_hbm, z_hbm, out_hbm,
           acc, src_v, dst_v, rows, ad2t, mt, sem1):
        cid = lax.axis_index("c")
        sid = lax.axis_index("s")
        wid = sid * 2 + cid
        base = wid * _EPT

        pltpu.sync_copy(z_hbm.at[pl.ds(sid * _RPT, _RPT)],
                        acc.at[pl.ds(sid * _RPT, _RPT)])
        pltpu.sync_copy(mt_hbm, mt)
        pltpu.sync_copy(ad2_hbm, ad2t)
        plsc.subcore_barrier()

        mtv = mt[...]
        m2 = mtv[0] + mtv[1]
        m2 = jnp.where(m2 > 0, m2, NEG * m2)
        m2v = _splat(m2)

        def body(i, _):
            eb = base + i * _B2
            pltpu.sync_copy(src_hbm.at[pl.ds(eb, _B2)], src_v)
            pltpu.sync_copy(dst_hbm.at[pl.ds(eb, _B2)], dst_v)
            pltpu.async_copy(xw_hbm.at[src_v], rows, sem1).wait()
            for j in range(_B2 // 16):
                e16 = _splat(j * 16, jnp.int32) + _iota16()
                dstv = dst_v[pl.ds(j * 16, 16)]
                asv = plsc.load_gather(rows, [e16, _splat(41, jnp.int32)])
                adv = plsc.load_gather(ad2t, [dstv])
                al = asv + adv
                al = jnp.where(al > 0, al, NEG * al)
                w = jnp.exp(al - m2v)
                for e in range(16):
                    ws = jnp.full((16,), w[e], jnp.float32)
                    for k in range(CB2S // 16):
                        val = rows[j * 16 + e, pl.ds(k * 16, 16)] * ws
                        rows[j * 16 + e, pl.ds(k * 16, 16)] = val
            pltpu.sync_copy(rows, acc.at[dst_v], add=True)
            return 0

        lax.fori_loop(0, _EPT // _B2, body, 0)
        plsc.subcore_barrier()
        pltpu.sync_copy(acc.at[pl.ds(sid * _RPT, _RPT)],
                        out_hbm.at[cid].at[pl.ds(sid * _RPT, _RPT)])

    return k7(xw2_ext, ad2, maxT2.reshape(16), src, dst, zeros_cb2)


_CH = 1024
_NCH = NP // _CH
_ACC1 = _CH + 8
_RPC = _CH // 16


def _edges_l1_sc(xw_t, ad1, maxT, pk, zeros1):
    mesh = plsc.VectorSubcoreMesh(core_axis_name="c", subcore_axis_name="s")

    @functools.partial(
        pl.kernel,
        out_type=jax.ShapeDtypeStruct((2, NTILE1, NP, HID), jnp.float32),
        mesh=mesh,
        compiler_params=pltpu.CompilerParams(needs_layout_passes=False),
        scratch_types=[
            pltpu.VMEM_SHARED((NTILE1, _ACC1, HID), jnp.float32),
            pltpu.VMEM((_EPT,), jnp.int32),
            pltpu.VMEM((_EPT + 16,), jnp.int32),
            pltpu.VMEM((NTILE1, 16, HID), jnp.float32),
            pltpu.VMEM((_ACC1 * HEADS,), jnp.float32),
            pltpu.VMEM((16,), jnp.float32),
            pltpu.VMEM((16,), jnp.int32),
            pltpu.SemaphoreType.DMA,
        ],
    )
    def k3(xw_hbm, ad1_hbm, mt_hbm, pk_hbm, z_hbm, out_hbm,
           acc, pkt, sel, rows, adt, mt, dlb, sem):
        cid = lax.axis_index("c")
        sid = lax.axis_index("s")
        wid = sid * 2 + cid
        base = wid * _EPT

        pltpu.sync_copy(pk_hbm.at[pl.ds(base, _EPT)], pkt)
        pltpu.sync_copy(mt_hbm, mt)
        mtv = mt[...]
        ms = []
        for h in range(HEADS):
            m = mtv[h] + mtv[HEADS + h]
            m = jnp.where(m > 0, m, NEG * m)
            ms.append(_splat(m))
        e16 = _iota16()

        def pass_body(cch, _):
            c0 = cch * _CH
            for t in range(NTILE1):
                pltpu.sync_copy(z_hbm.at[pl.ds(sid * _RPC, _RPC)],
                                acc.at[t].at[pl.ds(sid * _RPC, _RPC)])
            pltpu.sync_copy(ad1_hbm.at[pl.ds(c0 * HEADS, _CH * HEADS)],
                            adt.at[pl.ds(0, _CH * HEADS)])
            plsc.subcore_barrier()

            c0v = _splat(c0, jnp.int32)

            def scan_g(g, curv):
                pkv = pkt[pl.ds(g * 16, 16)]
                dloc = jnp.right_shift(pkv, 14) - c0v
                inb = (dloc >= 0) & (dloc < _CH)
                pf = plsc.cumsum(jnp.where(inb, 1, 0).astype(jnp.int32))
                pos = curv + pf - 1
                plsc.store_scatter(sel, [pos], pkv, mask=inb)
                return curv + plsc.all_reduce_population_count(inb)

            curv = lax.fori_loop(0, _EPT // 16, scan_g,
                                 jnp.zeros((16,), jnp.int32))
            padv = _splat((_CH << 14), jnp.int32) + c0v * 16384
            plsc.store_scatter(sel, [curv + e16], padv)
            cnt = curv[0]
            nb = (cnt + 15) // 16

            def proc(b, _):
                pv = sel[pl.ds(b * 16, 16)]
                sv = jnp.bitwise_and(pv, 16383)
                dl = jnp.right_shift(pv, 14) - c0v
                cps = [pltpu.async_copy(xw_hbm.at[t].at[sv], rows.at[t], sem)
                       for t in range(NTILE1)]
                for cp in cps:
                    cp.wait()
                whl = []
                for h in range(HEADS):
                    asv = plsc.load_gather(
                        rows, [_splat(HEADS, jnp.int32), e16,
                               _splat(8 + h, jnp.int32)])
                    adv = plsc.load_gather(
                        adt, [dl * HEADS + _splat(h, jnp.int32)])
                    al = asv + adv
                    al = jnp.where(al > 0, al, NEG * al)
                    whl.append(jnp.exp(al - ms[h]))
                for e in range(16):
                    for t in range(HEADS):
                        ws = jnp.full((16,), whl[t][e], jnp.float32)
                        for k in range(HID // 16):
                            val = rows[t, e, pl.ds(k * 16, 16)] * ws
                            rows[t, e, pl.ds(k * 16, 16)] = val
                t8 = _splat(HEADS, jnp.int32)
                for h in range(HEADS):
                    cv = _splat(h, jnp.int32)
                    val = plsc.load_gather(rows, [t8, e16, cv]) * whl[h]
                    plsc.store_scatter(rows, [t8, e16, cv], val)
                dlb[...] = dl
                for t in range(NTILE1):
                    pltpu.sync_copy(rows.at[t], acc.at[t].at[dlb], add=True)
                return 0

            lax.fori_loop(0, nb, proc, 0)
            plsc.subcore_barrier()
            for t in range(NTILE1):
                pltpu.sync_copy(
                    acc.at[t].at[pl.ds(sid * _RPC, _RPC)],
                    out_hbm.at[cid].at[t].at[pl.ds(c0 + sid * _RPC, _RPC)])
            plsc.subcore_barrier()
            return 0

        lax.fori_loop(0, _NCH, pass_body, 0)

    return k3(xw_t, ad1, maxT.reshape(16), pk, zeros1)


def kernel(x, edge_index, W1, att_src1, att_dst1, bias1,
           W2, att_src2, att_dst2, bias2):
    src = edge_index[0]
    dst = edge_index[1]

    A1 = jnp.zeros((HEADS * HID, 16), jnp.float32)
    for h in range(HEADS):
        A1 = A1.at[h * HID:(h + 1) * HID, h].set(att_src1[h])
        A1 = A1.at[h * HID:(h + 1) * HID, HEADS + h].set(att_dst1[h])
    W2e = jnp.zeros((HEADS * HID, CB2), jnp.float32).at[:, :CLASSES].set(W2)
    W2r = W2e.reshape(HEADS, HID, CB2)
    A2 = jnp.zeros((CB2, 16), jnp.float32)
    A2 = A2.at[:CLASSES, 0].set(att_src2[0])
    A2 = A2.at[:CLASSES, 1].set(att_dst2[0])
    bias1r = bias1.reshape(HEADS, HID)
    bias2p = jnp.zeros((1, CB2), jnp.float32).at[0, :CLASSES].set(bias2)

    xw_ext, T1, maxT = _k1(x, W1, A1)
    ad1 = jnp.zeros((NP, HEADS), jnp.float32).at[:N].set(
        T1[:, HEADS:]).reshape(-1)
    z1 = jnp.zeros((_CH, HID), jnp.float32)
    pk = jnp.bitwise_or(src.astype(jnp.int32),
                        jnp.left_shift(dst.astype(jnp.int32), 14))
    num_p = _edges_l1_sc(xw_ext, ad1, maxT, pk, z1)
    h_arr = _k4(num_p, xw_ext, T1, maxT, bias1r)
    xw2_ext, T2, maxT2 = _k5(h_arr, W2r, A2)
    z2 = jnp.zeros((NP, CB2), jnp.float32)
    ad2 = jnp.zeros((NP,), jnp.float32).at[:N].set(T2[:, 1])
    num2_p = _edges_l2_sc(xw2_ext, ad2, maxT2, src, dst, z2)
    return _k8(num2_p, xw2_ext, T2, maxT2, bias2p)
